# baseline (speedup 1.0000x reference)
import functools

import numpy as np
import jax
import jax.numpy as jnp
from jax import lax
from jax.experimental import pallas as pl
from jax.experimental.pallas import tpu as pltpu
from jax.experimental.pallas import tpu_sc as plsc

_HID = 128
_G = 256
_NPATCH = 60
_NNODE = 960
_COUT = 132
_CHUNK_W = 240 * _COUT
_STAGE_TAILS = 960
_STAGE_PV = 15 * _HID
_NW = 32
_GPW = _G // _NW
_UNROLL = 12


def _build_tables():
    perm = np.zeros(60, np.int32)
    for q in range(4):
        for u in range(15):
            perm[q * 15 + u] = 4 * u + q
    w = np.arange(_CHUNK_W, dtype=np.int64)
    n_local = w // _COUT
    c = w % _COUT
    u = (n_local % 60) // 4
    tbl = np.where(c < _HID,
                   _STAGE_TAILS + u * _HID + c,
                   n_local * 4 + (c - _HID)).astype(np.int32)
    r = np.arange(4)[:, None, None]
    uu = np.arange(15)[None, :, None]
    v = np.arange(4)[None, None, :]
    per_q = []
    for q in range(4):
        ch = np.stack(np.broadcast_arrays(
            uu, np.full_like(uu + r + v, q), 4 * uu + v, 4 * q + r),
            axis=-1)
        per_q.append(ch.reshape(-1))
    tails = np.asarray(per_q, dtype=np.float32)
    return perm, tbl, tails


_PERM, _TBL, _TAILS = _build_tables()


def _mlp(x_ref, w1_ref, b1_ref, w2_ref, b2_ref, pv_ref):
    x = x_ref[...]
    h = jnp.maximum(
        jnp.dot(x, w1_ref[...], preferred_element_type=jnp.float32)
        + b1_ref[...], 0.0)
    pv_ref[...] = (jnp.dot(h, w2_ref[...], preferred_element_type=jnp.float32)
                   + b2_ref[...])


def _sc_expand(pv_hbm, tails_hbm, tbl_hbm, out_hbm,
               tbl_v, stage_v, img0, img1, sem0, sem1):
    wid = lax.axis_index("s") * 2 + lax.axis_index("c")
    pltpu.sync_copy(tbl_hbm, tbl_v)
    imgs = (img0, img1)
    sems = (sem0, sem1)
    pending = [None, None]

    def assemble(k, img):
        base = k * (_UNROLL * 16)
        for t in range(_UNROLL):
            off = base + t * 16
            idx = tbl_v[pl.ds(off, 16)]
            img[pl.ds(off, 16)] = plsc.load_gather(stage_v, [idx])
        return k + 1

    ci = 0
    for j in range(_GPW):
        g = wid * _GPW + j
        for q in range(4):
            buf = ci % 2
            img, sem = imgs[buf], sems[buf]
            pltpu.sync_copy(tails_hbm.at[pl.ds(q * _STAGE_TAILS,
                                               _STAGE_TAILS)],
                            stage_v.at[pl.ds(0, _STAGE_TAILS)])
            pltpu.sync_copy(
                pv_hbm.at[pl.ds(g * 4 * _STAGE_PV + q * _STAGE_PV,
                                _STAGE_PV)],
                stage_v.at[pl.ds(_STAGE_TAILS, _STAGE_PV)])
            if pending[buf] is not None:
                pending[buf].wait()
                pending[buf] = None
            lax.fori_loop(0, _CHUNK_W // (_UNROLL * 16),
                          lambda k, _, img=img: (assemble(k, img), _)[1], 0)
            chunk_base = (g * 4 + q) * _CHUNK_W
            pending[buf] = pltpu.async_copy(
                img, out_hbm.at[pl.ds(chunk_base, _CHUNK_W)], sem)
            ci += 1
    for b in range(2):
        if pending[b] is not None:
            pending[b].wait()


def kernel(patch_vectors, W1, b1, W2, b2, Wg1, bg1, Wg2, bg2, mesh_edges):
    del Wg1, bg1, Wg2, bg2, mesh_edges
    x_perm = (patch_vectors.reshape(_G, _NPATCH, 3)[:, _PERM, :]
              .reshape(_G * _NPATCH, 3))
    rows_blk = 1920
    pv = pl.pallas_call(
        _mlp,
        grid=(_G * _NPATCH // rows_blk,),
        in_specs=[
            pl.BlockSpec((rows_blk, 3), lambda i: (i, 0)),
            pl.BlockSpec((3, _HID), lambda i: (0, 0)),
            pl.BlockSpec((1, _HID), lambda i: (0, 0)),
            pl.BlockSpec((_HID, _HID), lambda i: (0, 0)),
            pl.BlockSpec((1, _HID), lambda i: (0, 0)),
        ],
        out_specs=pl.BlockSpec((rows_blk, _HID), lambda i: (i, 0)),
        out_shape=jax.ShapeDtypeStruct((_G * _NPATCH, _HID), jnp.float32),
    )(x_perm, W1, b1.reshape(1, _HID), W2, b2.reshape(1, _HID))

    expand = functools.partial(
        pl.kernel,
        out_type=jax.ShapeDtypeStruct((_G * _NNODE * _COUT,), jnp.float32),
        mesh=plsc.VectorSubcoreMesh(core_axis_name="c", subcore_axis_name="s"),
        scratch_types=[
            pltpu.VMEM((_CHUNK_W,), jnp.int32),
            pltpu.VMEM((_STAGE_TAILS + _STAGE_PV,), jnp.float32),
            pltpu.VMEM((_CHUNK_W,), jnp.float32),
            pltpu.VMEM((_CHUNK_W,), jnp.float32),
            pltpu.SemaphoreType.DMA,
            pltpu.SemaphoreType.DMA,
        ],
        compiler_params=pltpu.CompilerParams(needs_layout_passes=False),
    )(_sc_expand)
    out = expand(pv.reshape(_G * _NPATCH * _HID),
                 jnp.asarray(_TAILS.reshape(-1)),
                 jnp.asarray(_TBL))
    return out.reshape(_G, _NNODE, _COUT)


# speedup vs baseline: 1.6458x; 1.6458x over previous
import functools

import numpy as np
import jax
import jax.numpy as jnp
from jax import lax
from jax.experimental import pallas as pl
from jax.experimental.pallas import tpu as pltpu
from jax.experimental.pallas import tpu_sc as plsc

_HID = 128
_G = 256
_NPATCH = 60
_NNODE = 960
_COUT = 132
_CHUNK_W = 240 * _COUT
_STAGE_TAILS = 960
_STAGE_PV = 15 * _HID
_NW = 32
_GPW = _G // _NW
_UNROLL = 12


def _build_tables():
    perm = np.zeros(60, np.int32)
    for q in range(4):
        for u in range(15):
            perm[q * 15 + u] = 4 * u + q
    w = np.arange(_CHUNK_W, dtype=np.int64)
    n_local = w // _COUT
    c = w % _COUT
    u = (n_local % 60) // 4
    tbl = np.where(c < _HID,
                   _STAGE_TAILS + u * _HID + c,
                   n_local * 4 + (c - _HID)).astype(np.int32)
    r = np.arange(4)[:, None, None]
    uu = np.arange(15)[None, :, None]
    v = np.arange(4)[None, None, :]
    per_q = []
    for q in range(4):
        ch = np.stack(np.broadcast_arrays(
            uu, np.full_like(uu + r + v, q), 4 * uu + v, 4 * q + r),
            axis=-1)
        per_q.append(ch.reshape(-1))
    tails = np.asarray(per_q, dtype=np.float32)
    return perm, tbl, tails


_PERM, _TBL, _TAILS = _build_tables()


def _mlp(x_ref, w1_ref, b1_ref, w2_ref, b2_ref, pv_ref):
    x = x_ref[...]
    h = jnp.maximum(
        jnp.dot(x, w1_ref[...], preferred_element_type=jnp.float32)
        + b1_ref[...], 0.0)
    pv_ref[...] = (jnp.dot(h, w2_ref[...], preferred_element_type=jnp.float32)
                   + b2_ref[...])


def _sc_expand(pv_hbm, tails_hbm, tbl_hbm, out_hbm,
               tbl_v, stage_v, img0, img1, sem0, sem1):
    wid = lax.axis_index("s") * 2 + lax.axis_index("c")
    pltpu.sync_copy(tbl_hbm, tbl_v)
    imgs = (img0, img1)
    sems = (sem0, sem1)
    pending = [None, None]

    def assemble(img):
        @plsc.parallel_loop(0, _CHUNK_W, 16, unroll=_UNROLL)
        def _(off):
            idx = tbl_v[pl.ds(off, 16)]
            img[pl.ds(off, 16)] = plsc.load_gather(stage_v, [idx])

    ci = 0
    for j in range(_GPW):
        g = wid * _GPW + j
        for q in range(4):
            buf = ci % 2
            img, sem = imgs[buf], sems[buf]
            pltpu.sync_copy(tails_hbm.at[pl.ds(q * _STAGE_TAILS,
                                               _STAGE_TAILS)],
                            stage_v.at[pl.ds(0, _STAGE_TAILS)])
            pltpu.sync_copy(
                pv_hbm.at[pl.ds(g * 4 * _STAGE_PV + q * _STAGE_PV,
                                _STAGE_PV)],
                stage_v.at[pl.ds(_STAGE_TAILS, _STAGE_PV)])
            if pending[buf] is not None:
                pending[buf].wait()
                pending[buf] = None
            assemble(img)
            chunk_base = (g * 4 + q) * _CHUNK_W
            pending[buf] = pltpu.async_copy(
                img, out_hbm.at[pl.ds(chunk_base, _CHUNK_W)], sem)
            ci += 1
    for b in range(2):
        if pending[b] is not None:
            pending[b].wait()


def kernel(patch_vectors, W1, b1, W2, b2, Wg1, bg1, Wg2, bg2, mesh_edges):
    del Wg1, bg1, Wg2, bg2, mesh_edges
    x_perm = (patch_vectors.reshape(_G, _NPATCH, 3)[:, _PERM, :]
              .reshape(_G * _NPATCH, 3))
    rows_blk = 1920
    pv = pl.pallas_call(
        _mlp,
        grid=(_G * _NPATCH // rows_blk,),
        in_specs=[
            pl.BlockSpec((rows_blk, 3), lambda i: (i, 0)),
            pl.BlockSpec((3, _HID), lambda i: (0, 0)),
            pl.BlockSpec((1, _HID), lambda i: (0, 0)),
            pl.BlockSpec((_HID, _HID), lambda i: (0, 0)),
            pl.BlockSpec((1, _HID), lambda i: (0, 0)),
        ],
        out_specs=pl.BlockSpec((rows_blk, _HID), lambda i: (i, 0)),
        out_shape=jax.ShapeDtypeStruct((_G * _NPATCH, _HID), jnp.float32),
    )(x_perm, W1, b1.reshape(1, _HID), W2, b2.reshape(1, _HID))

    expand = functools.partial(
        pl.kernel,
        out_type=jax.ShapeDtypeStruct((_G * _NNODE * _COUT,), jnp.float32),
        mesh=plsc.VectorSubcoreMesh(core_axis_name="c", subcore_axis_name="s"),
        scratch_types=[
            pltpu.VMEM((_CHUNK_W,), jnp.int32),
            pltpu.VMEM((_STAGE_TAILS + _STAGE_PV,), jnp.float32),
            pltpu.VMEM((_CHUNK_W,), jnp.float32),
            pltpu.VMEM((_CHUNK_W,), jnp.float32),
            pltpu.SemaphoreType.DMA,
            pltpu.SemaphoreType.DMA,
        ],
        compiler_params=pltpu.CompilerParams(needs_layout_passes=False),
    )(_sc_expand)
    out = expand(pv.reshape(_G * _NPATCH * _HID),
                 jnp.asarray(_TAILS.reshape(-1)),
                 jnp.asarray(_TBL))
    return out.reshape(_G, _NNODE, _COUT)


# speedup vs baseline: 1.7599x; 1.0693x over previous
import functools

import numpy as np
import jax
import jax.numpy as jnp
from jax import lax
from jax.experimental import pallas as pl
from jax.experimental.pallas import tpu as pltpu
from jax.experimental.pallas import tpu_sc as plsc

_HID = 128
_G = 256
_NPATCH = 60
_NNODE = 960
_COUT = 132
_CHUNK_W = 240 * _COUT
_STAGE_TAILS = 960
_STAGE_PV = 15 * _HID
_NW = 32
_GPW = _G // _NW
_UNROLL = 12


def _build_tables():
    perm = np.zeros(60, np.int32)
    for q in range(4):
        for u in range(15):
            perm[q * 15 + u] = 4 * u + q
    w = np.arange(_CHUNK_W, dtype=np.int64)
    n_local = w // _COUT
    c = w % _COUT
    u = (n_local % 60) // 4
    tbl = np.where(c < _HID,
                   _STAGE_TAILS + u * _HID + c,
                   n_local * 4 + (c - _HID)).astype(np.int32)
    r = np.arange(4)[:, None, None]
    uu = np.arange(15)[None, :, None]
    v = np.arange(4)[None, None, :]
    per_q = []
    for q in range(4):
        ch = np.stack(np.broadcast_arrays(
            uu, np.full_like(uu + r + v, q), 4 * uu + v, 4 * q + r),
            axis=-1)
        per_q.append(ch.reshape(-1))
    tails = np.asarray(per_q, dtype=np.float32)
    return perm, tbl, tails


_PERM, _TBL, _TAILS = _build_tables()


def _mlp(x_ref, w1_ref, b1_ref, w2_ref, b2_ref, pv_ref):
    x = x_ref[...]
    h = jnp.maximum(
        jnp.dot(x, w1_ref[...], preferred_element_type=jnp.float32)
        + b1_ref[...], 0.0)
    pv_ref[...] = (jnp.dot(h, w2_ref[...], preferred_element_type=jnp.float32)
                   + b2_ref[...])


def _sc_expand(pv_hbm, tails_hbm, tbl_hbm, out_hbm,
               tbl_v, stage0, stage1, img0, img1,
               sem_s0, sem_s1, sem0, sem1):
    wid = lax.axis_index("s") * 2 + lax.axis_index("c")
    pltpu.sync_copy(tbl_hbm, tbl_v)
    imgs = (img0, img1)
    sems = (sem0, sem1)
    stages = (stage0, stage1)
    stage_sems = (sem_s0, sem_s1)
    pending = [None, None]
    stage_pending = [None, None]

    _OFFS = (0, 16, 32, 48, 64, 80, 96, 112, 116)

    def assemble(img, stage):
        @plsc.parallel_loop(0, 240, 1, unroll=2)
        def _(r):
            base = r * _COUT
            for o in _OFFS:
                idx = tbl_v[pl.ds(base + o, 16)]
                img[r, pl.ds(o, 16)] = plsc.load_gather(stage, [idx])

    def issue_stage(ci):
        g = wid * _GPW + ci // 4
        q = ci % 4
        buf = ci % 2
        stage, ssem = stages[buf], stage_sems[buf]
        h1 = pltpu.async_copy(
            tails_hbm.at[pl.ds(q * _STAGE_TAILS, _STAGE_TAILS)],
            stage.at[pl.ds(0, _STAGE_TAILS)], ssem)
        h2 = pltpu.async_copy(
            pv_hbm.at[pl.ds(g * 4 * _STAGE_PV + q * _STAGE_PV, _STAGE_PV)],
            stage.at[pl.ds(_STAGE_TAILS, _STAGE_PV)], ssem)
        stage_pending[buf] = (h1, h2)

    n_chunks = _GPW * 4
    issue_stage(0)
    for ci in range(n_chunks):
        buf = ci % 2
        img, sem = imgs[buf], sems[buf]
        for h in stage_pending[buf]:
            h.wait()
        stage_pending[buf] = None
        if ci + 1 < n_chunks:
            issue_stage(ci + 1)
        if pending[buf] is not None:
            pending[buf].wait()
            pending[buf] = None
        assemble(img, stages[buf])
        g = wid * _GPW + ci // 4
        pending[buf] = pltpu.async_copy(
            img, out_hbm.at[g, pl.ds((ci % 4) * 240, 240), :], sem)
    for b in range(2):
        if pending[b] is not None:
            pending[b].wait()


def kernel(patch_vectors, W1, b1, W2, b2, Wg1, bg1, Wg2, bg2, mesh_edges):
    del Wg1, bg1, Wg2, bg2, mesh_edges
    x_perm = (patch_vectors.reshape(_G, _NPATCH, 3)[:, _PERM, :]
              .reshape(_G * _NPATCH, 3))
    rows_blk = 1920
    pv = pl.pallas_call(
        _mlp,
        grid=(_G * _NPATCH // rows_blk,),
        in_specs=[
            pl.BlockSpec((rows_blk, 3), lambda i: (i, 0)),
            pl.BlockSpec((3, _HID), lambda i: (0, 0)),
            pl.BlockSpec((1, _HID), lambda i: (0, 0)),
            pl.BlockSpec((_HID, _HID), lambda i: (0, 0)),
            pl.BlockSpec((1, _HID), lambda i: (0, 0)),
        ],
        out_specs=pl.BlockSpec((rows_blk, _HID), lambda i: (i, 0)),
        out_shape=jax.ShapeDtypeStruct((_G * _NPATCH, _HID), jnp.float32),
    )(x_perm, W1, b1.reshape(1, _HID), W2, b2.reshape(1, _HID))

    expand = functools.partial(
        pl.kernel,
        out_type=jax.ShapeDtypeStruct((_G, _NNODE, _COUT), jnp.float32),
        mesh=plsc.VectorSubcoreMesh(core_axis_name="c", subcore_axis_name="s"),
        scratch_types=[
            pltpu.VMEM((_CHUNK_W,), jnp.int32),
            pltpu.VMEM((_STAGE_TAILS + _STAGE_PV,), jnp.float32),
            pltpu.VMEM((_STAGE_TAILS + _STAGE_PV,), jnp.float32),
            pltpu.VMEM((240, _COUT), jnp.float32),
            pltpu.VMEM((240, _COUT), jnp.float32),
            pltpu.SemaphoreType.DMA,
            pltpu.SemaphoreType.DMA,
            pltpu.SemaphoreType.DMA,
            pltpu.SemaphoreType.DMA,
        ],
        compiler_params=pltpu.CompilerParams(needs_layout_passes=False,
                                             use_tc_tiling_on_sc=False),
    )(_sc_expand)
    out = expand(pv.reshape(_G * _NPATCH * _HID),
                 jnp.asarray(_TAILS.reshape(-1)),
                 jnp.asarray(_TBL))
    return out


# speedup vs baseline: 4.3745x; 2.4857x over previous
import numpy as np
import jax
import jax.numpy as jnp
from jax.experimental import pallas as pl
from jax.experimental.pallas import tpu as pltpu

_HID = 128
_NPATCH = 60
_NNODE = 960
_COUT = 132
_G = 256


def _build_idx4():
    ny, nx = np.meshgrid(np.arange(16), np.arange(60), indexing="ij")
    a = np.stack([nx // 4, ny // 4, nx, ny], axis=-1).astype(np.float32)
    return a.reshape(4, 4, 15, 4, 4)


def _fused(x_ref, w1_ref, b1_ref, w2_ref, b2_ref, idx_ref, out_ref, *, gb):
    x = x_ref[...]
    h = jnp.maximum(
        jnp.dot(x, w1_ref[...], preferred_element_type=jnp.float32)
        + b1_ref[...], 0.0)
    pv = (jnp.dot(h, w2_ref[...], preferred_element_type=jnp.float32)
          + b2_ref[...])
    pv = pv.reshape(gb, 15, 4, _HID).transpose(0, 2, 1, 3)
    t = jnp.broadcast_to(pv[:, :, None, :, None, :],
                         (gb, 4, 4, 15, 4, _HID))
    out_ref[..., : _HID] = t.reshape(gb, _NNODE, _HID)
    out_ref[..., _HID: _HID + 4] = jnp.broadcast_to(
        idx_ref[...].reshape(1, _NNODE, 4), (gb, _NNODE, 4))


def kernel(patch_vectors, W1, b1, W2, b2, Wg1, bg1, Wg2, bg2, mesh_edges):
    del Wg1, bg1, Wg2, bg2, mesh_edges
    gb = 8
    grid = _G // gb
    x = patch_vectors.reshape(_G * _NPATCH, 3)
    idx4 = jnp.asarray(_build_idx4())
    out6 = pl.pallas_call(
        lambda *refs: _fused(*refs, gb=gb),
        grid=(grid,),
        in_specs=[
            pl.BlockSpec((gb * _NPATCH, 3), lambda i: (i, 0)),
            pl.BlockSpec((3, _HID), lambda i: (0, 0)),
            pl.BlockSpec((1, _HID), lambda i: (0, 0)),
            pl.BlockSpec((_HID, _HID), lambda i: (0, 0)),
            pl.BlockSpec((1, _HID), lambda i: (0, 0)),
            pl.BlockSpec((4, 4, 15, 4, 4), lambda i: (0, 0, 0, 0, 0)),
        ],
        out_specs=pl.BlockSpec((gb, _NNODE, 256),
                               lambda i: (i, 0, 0)),
        out_shape=jax.ShapeDtypeStruct((_G, _NNODE, 256), jnp.float32),
    )(x, W1, b1.reshape(1, _HID), W2, b2.reshape(1, _HID), idx4)
    return out6[..., : _COUT]


# speedup vs baseline: 4.3761x; 1.0004x over previous
import numpy as np
import jax
import jax.numpy as jnp
from jax.experimental import pallas as pl
from jax.experimental.pallas import tpu as pltpu

_HID = 128
_NPATCH = 60
_NNODE = 960
_COUT = 132
_G = 256


def _build_idx4():
    ny, nx = np.meshgrid(np.arange(16), np.arange(60), indexing="ij")
    a = np.stack([nx // 4, ny // 4, nx, ny], axis=-1).astype(np.float32)
    return a.reshape(4, 4, 15, 4, 4)


def _fused(x_ref, w1_ref, b1_ref, w2_ref, b2_ref, idx_ref, out_ref, *, gb):
    x = x_ref[...]
    h = jnp.maximum(
        jnp.dot(x, w1_ref[...], preferred_element_type=jnp.float32)
        + b1_ref[...], 0.0)
    pv = (jnp.dot(h, w2_ref[...], preferred_element_type=jnp.float32)
          + b2_ref[...])
    pv = pv.reshape(gb, 15, 4, _HID).transpose(0, 2, 1, 3)
    t = jnp.broadcast_to(pv[:, :, None, :, None, :],
                         (gb, 4, 4, 15, 4, _HID))
    out_ref[..., : _HID] = t.reshape(gb, _NNODE, _HID)
    out_ref[..., _HID: _HID + 4] = jnp.broadcast_to(
        idx_ref[...].reshape(1, _NNODE, 4), (gb, _NNODE, 4))


def kernel(patch_vectors, W1, b1, W2, b2, Wg1, bg1, Wg2, bg2, mesh_edges):
    del Wg1, bg1, Wg2, bg2, mesh_edges
    gb = 16
    grid = _G // gb
    x = patch_vectors.reshape(_G * _NPATCH, 3)
    idx4 = jnp.asarray(_build_idx4())
    out6 = pl.pallas_call(
        lambda *refs: _fused(*refs, gb=gb),
        grid=(grid,),
        in_specs=[
            pl.BlockSpec((gb * _NPATCH, 3), lambda i: (i, 0)),
            pl.BlockSpec((3, _HID), lambda i: (0, 0)),
            pl.BlockSpec((1, _HID), lambda i: (0, 0)),
            pl.BlockSpec((_HID, _HID), lambda i: (0, 0)),
            pl.BlockSpec((1, _HID), lambda i: (0, 0)),
            pl.BlockSpec((4, 4, 15, 4, 4), lambda i: (0, 0, 0, 0, 0)),
        ],
        out_specs=pl.BlockSpec((gb, _NNODE, 256),
                               lambda i: (i, 0, 0)),
        out_shape=jax.ShapeDtypeStruct((_G, _NNODE, 256), jnp.float32),
    )(x, W1, b1.reshape(1, _HID), W2, b2.reshape(1, _HID), idx4)
    return out6[..., : _COUT]
